# baseline (speedup 1.0000x reference)
import functools

import jax
import jax.numpy as jnp
from jax import lax
from jax.experimental import pallas as pl
from jax.experimental.pallas import tpu as pltpu
from jax.experimental.pallas import tpu_sc as plsc

_F32 = jnp.float32
_BIG = 1e30


def _embed_t1_kernel(x_ref, wep_ref, be_ref, w1af_ref, w1axp_ref, b1a_ref,
                     f0_ref, t1_ref):
    xb = x_ref[0]
    f0 = jax.nn.relu(
        jnp.dot(xb, wep_ref[...], preferred_element_type=_F32) + be_ref[...])
    t1 = (jnp.dot(f0, w1af_ref[...], preferred_element_type=_F32)
          + jnp.dot(xb, w1axp_ref[...], preferred_element_type=_F32)
          + b1a_ref[...])
    f0_ref[0] = f0
    t1_ref[0] = t1


def _knn_dist_kernel(k, csz, nq_ref, pt_ref, xp_ref, nqt_ref, d2_ref, cid_ref):
    pt = pt_ref[0]
    pp = jnp.sum(pt * pt, axis=0, keepdims=True)
    d2 = jnp.dot(-2.0 * nq_ref[0], pt, preferred_element_type=_F32) + pp
    d2_ref[0] = d2

    xb = xp_ref[0]
    lane8 = lax.broadcasted_iota(jnp.int32, xb.shape, 1)
    ppt = jnp.sum(jnp.where(lane8 < 3, xb * xb, 0.0), axis=1, keepdims=True)
    d2t = jnp.dot(xb, -2.0 * nqt_ref[0], preferred_element_type=_F32) + ppt
    n, qb = d2t.shape
    c = n // csz
    cm = jnp.min(d2t.reshape(c, csz, qb), axis=1)

    sub_c = lax.broadcasted_iota(jnp.int32, (c, qb), 0)
    sub_k = lax.broadcasted_iota(jnp.int32, (k, qb), 0)

    def body(r, carry):
        cmc, acc = carry
        m = jnp.min(cmc, axis=0, keepdims=True)
        first = jnp.min(jnp.where(cmc <= m, sub_c, c), axis=0, keepdims=True)
        acc = jnp.where(sub_k == r, first, acc)
        cmc = jnp.where(sub_c == first, _BIG, cmc)
        return cmc, acc

    _, cid = lax.fori_loop(0, k, body, (cm, jnp.zeros((k, qb), jnp.int32)))
    cid_ref[0] = cid


def _knn_select_kernel(k, csz, n, cand_ref, cid_ref, e_ref, idx_ref):
    cand = cand_ref[0]
    q, w = cand.shape
    base = jnp.dot(cid_ref[0].astype(_F32), e_ref[...],
                   preferred_element_type=_F32)
    iota_w = lax.broadcasted_iota(jnp.int32, (q, w), 1)
    orig = base.astype(jnp.int32) * csz + jnp.remainder(iota_w, csz)
    lane_k = lax.broadcasted_iota(jnp.int32, (q, k), 1)

    def body(r, carry):
        d2c, acc = carry
        m = jnp.min(d2c, axis=1, keepdims=True)
        first = jnp.min(jnp.where(d2c <= m, orig, n), axis=1, keepdims=True)
        acc = jnp.where(lane_k == r, first, acc)
        d2c = jnp.where(orig == first, _BIG, d2c)
        return d2c, acc

    _, acc = lax.fori_loop(0, k, body, (cand, jnp.zeros((q, k), jnp.int32)))
    idx_ref[0] = acc


def _gather_rows(table, idx):
    rows, depth = table.shape
    m = idx.shape[0]
    nw = 32
    ch = 128
    per_w = m // nw
    steps = per_w // ch
    mesh = plsc.VectorSubcoreMesh(core_axis_name="c", subcore_axis_name="s")

    @functools.partial(
        pl.kernel,
        mesh=mesh,
        out_type=jax.ShapeDtypeStruct((m, depth), _F32),
        scratch_types=[
            pltpu.VMEM((ch,), jnp.int32),
            pltpu.VMEM((ch, depth), _F32),
            pltpu.SemaphoreType.DMA,
        ],
    )
    def gk(tab_hbm, idx_hbm, out_hbm, idx_v, rows_v, sem):
        wid = lax.axis_index("s") * 2 + lax.axis_index("c")
        base = wid * per_w

        def body(j, carry):
            off = base + j * ch
            pltpu.sync_copy(idx_hbm.at[pl.ds(off, ch)], idx_v)
            pltpu.async_copy(tab_hbm.at[idx_v], rows_v, sem).wait()
            pltpu.sync_copy(rows_v, out_hbm.at[pl.ds(off, ch)])
            return carry

        lax.fori_loop(0, steps, body, 0)

    return gk(table, idx)


def _sa1_finish_kernel(s1, k, g_ref, nq_ref, w1axp_ref, w1b_ref, b1b_ref,
                       w2af_ref, b2a_ref, w2axp_ref, f1_ref, t2_ref):
    g = g_ref[0]
    pq = jnp.dot(nq_ref[0], w1axp_ref[...], preferred_element_type=_F32)
    h1 = jax.nn.relu(g.reshape(s1, k, 128) - pq[:, None, :]).reshape(s1 * k, 128)
    h2 = jax.nn.relu(
        jnp.dot(h1, w1b_ref[...], preferred_element_type=_F32) + b1b_ref[...])
    f1 = jnp.max(h2.reshape(s1, k, 128), axis=1)
    t2 = (jnp.dot(f1, w2af_ref[...], preferred_element_type=_F32)
          + b2a_ref[...]
          + jnp.dot(nq_ref[0], w2axp_ref[...], preferred_element_type=_F32))
    f1_ref[0] = f1
    t2_ref[0] = t2


def _top3_weights(d2f):
    q, s = d2f.shape
    iota_n = lax.broadcasted_iota(jnp.int32, (q, s), 1)
    inv = 1.0 / jnp.maximum(jnp.sqrt(d2f), 1e-10)
    w = jnp.zeros((q, s), _F32)
    d2c = d2f
    for _ in range(3):
        m = jnp.min(d2c, axis=1, keepdims=True)
        first = jnp.min(jnp.where(d2c <= m, iota_n, s), axis=1, keepdims=True)
        sel = iota_n == first
        w = jnp.where(sel, inv, w)
        d2c = jnp.where(sel, _BIG, d2c)
    return w / jnp.sum(w, axis=1, keepdims=True)


def _sa2_fp2_kernel(s1, s2, k, x1p_ref, x1t_ref, x2p_ref, x2t_ref, t2_ref,
                    f1_ref, w2axp_ref, w2b_ref, b2b_ref, wf2at_ref, wf2ab_ref,
                    bf2a_ref, wf2b_ref, bf2b_ref, f1up_ref):
    x1t = x1t_ref[0]
    pp1 = jnp.sum(x1t * x1t, axis=0, keepdims=True)
    d2 = jnp.dot(-2.0 * x2p_ref[0], x1t, preferred_element_type=_F32) + pp1
    iota_n = lax.broadcasted_iota(jnp.int32, (s2, s1), 1)
    pq2 = jnp.dot(x2p_ref[0], w2axp_ref[...], preferred_element_type=_F32)

    rowmod = jnp.remainder(
        lax.broadcasted_iota(jnp.int32, (s2 * k, s1), 0), k)

    def body(r, carry):
        d2c, oh = carry
        m = jnp.min(d2c, axis=1, keepdims=True)
        first = jnp.min(jnp.where(d2c <= m, iota_n, s1), axis=1, keepdims=True)
        sel = iota_n == first
        d2c = jnp.where(sel, _BIG, d2c)
        sel3 = jnp.broadcast_to(sel[:, None, :], (s2, k, s1)).reshape(s2 * k, s1)
        oh = jnp.where((rowmod == r) & sel3, 1.0, oh)
        return d2c, oh

    _, oh = lax.fori_loop(
        0, k, body, (d2, jnp.zeros((s2 * k, s1), _F32)))
    g2 = jnp.dot(oh, t2_ref[0], preferred_element_type=_F32)
    pq2r = jnp.broadcast_to(pq2[:, None, :], (s2, k, 256)).reshape(s2 * k, 256)
    h1 = jax.nn.relu(g2 - pq2r)
    h2 = jax.nn.relu(
        jnp.dot(h1, w2b_ref[...], preferred_element_type=_F32) + b2b_ref[...])
    f2 = jnp.max(h2.reshape(s2, k, 256), axis=1)

    x2t = x2t_ref[0]
    cc2 = jnp.sum(x2t * x2t, axis=0, keepdims=True)
    x1p = x1p_ref[0]
    qq1 = jnp.sum(x1p * x1p, axis=1, keepdims=True)
    d2f = jnp.maximum(
        jnp.dot(x1p, -2.0 * x2t, preferred_element_type=_F32) + cc2 + qq1, 0.0)
    w = _top3_weights(d2f)
    interp = jnp.dot(w, f2, preferred_element_type=_F32)
    xcat = jax.nn.relu(
        jnp.dot(f1_ref[0], wf2at_ref[...], preferred_element_type=_F32)
        + jnp.dot(interp, wf2ab_ref[...], preferred_element_type=_F32)
        + bf2a_ref[...])
    f1up = jax.nn.relu(
        jnp.dot(xcat, wf2b_ref[...], preferred_element_type=_F32) + bf2b_ref[...])
    f1up_ref[0] = f1up


def _fp1_kernel(xp_ref, f0_ref, x1t_ref, f1up_ref, wf1at_ref, wf1ab_ref,
                bf1a_ref, wf1b_ref, bf1b_ref, f0up_ref, gf_ref):
    xb = xp_ref[0]
    lane8 = lax.broadcasted_iota(jnp.int32, xb.shape, 1)
    qq = jnp.sum(jnp.where(lane8 < 3, xb * xb, 0.0), axis=1, keepdims=True)
    x1t = x1t_ref[0]
    cc1 = jnp.sum(x1t * x1t, axis=0, keepdims=True)
    d2 = jnp.maximum(
        jnp.dot(xb, -2.0 * x1t, preferred_element_type=_F32) + cc1 + qq, 0.0)
    w = _top3_weights(d2)
    interp = jnp.dot(w, f1up_ref[0], preferred_element_type=_F32)
    x1 = jax.nn.relu(
        jnp.dot(f0_ref[0], wf1at_ref[...], preferred_element_type=_F32)
        + jnp.dot(interp, wf1ab_ref[...], preferred_element_type=_F32)
        + bf1a_ref[...])
    f0up = jax.nn.relu(
        jnp.dot(x1, wf1b_ref[...], preferred_element_type=_F32) + bf1b_ref[...])
    f0up_ref[0] = f0up
    bm = jnp.broadcast_to(jnp.max(f0up, axis=0, keepdims=True), (8, 128))[None]

    @pl.when(pl.program_id(1) == 0)
    def _():
        gf_ref[...] = bm

    @pl.when(pl.program_id(1) != 0)
    def _():
        gf_ref[...] = jnp.maximum(gf_ref[...], bm)


def _head_kernel(f0up_ref, gf_ref, wh1t_ref, wh1b_ref, bh1_ref, wh2_ref,
                 bh2_ref, wh3_ref, bh3_ref, out_ref):
    bias = (jnp.dot(gf_ref[0, 0:1, :], wh1b_ref[...],
                    preferred_element_type=_F32)
            + bh1_ref[...])
    h1 = jax.nn.relu(
        jnp.dot(f0up_ref[0], wh1t_ref[...], preferred_element_type=_F32) + bias)
    h2 = jax.nn.relu(
        jnp.dot(h1, wh2_ref[...], preferred_element_type=_F32) + bh2_ref[...])
    out_ref[0] = (jnp.dot(h2, wh3_ref[...], preferred_element_type=_F32)
                  + bh3_ref[...])


def _full(shape):
    return pl.BlockSpec(shape, lambda *_: tuple(0 for _ in shape))


def kernel(x, seed, We, be, W1a, b1a, W1b, b1b, W2a, b2a, W2b, b2b,
           Wf2a, bf2a, Wf2b, bf2b, Wf1a, bf1a, Wf1b, bf1b,
           Wh1, bh1, Wh2, bh2, Wh3, bh3):
    B, N, C = x.shape
    S1, S2, K = 256, 64, 32
    BLK = 2048 if N % 2048 == 0 else N
    QB = 128
    ncls = Wh3.shape[1]

    xp = jnp.pad(x, ((0, 0), (0, 0), (0, 8 - C)))
    xyz = x[..., :3]

    idx_s1 = jnp.stack([
        jax.random.permutation(jax.random.key(seed + b * 17), N)[:S1]
        for b in range(B)], 0)
    new_xyz = jnp.take_along_axis(xyz, idx_s1[..., None], axis=1)
    nq1 = jnp.pad(new_xyz, ((0, 0), (0, 0), (0, 5)))
    xyzT = jnp.pad(jnp.swapaxes(xyz, 1, 2), ((0, 0), (0, 5), (0, 0)))

    idx_s2 = jnp.stack([
        jax.random.permutation(jax.random.key(seed + 1000 + b * 17), S1)[:S2]
        for b in range(B)], 0)
    new_xyz2 = jnp.take_along_axis(new_xyz, idx_s2[..., None], axis=1)
    nq2 = jnp.pad(new_xyz2, ((0, 0), (0, 0), (0, 5)))
    x1t = jnp.swapaxes(nq1, 1, 2)
    x2t = jnp.swapaxes(nq2, 1, 2)

    wep = jnp.pad(We, ((0, 8 - C), (0, 0)))
    w1af = W1a[3:, :]
    w1axp = jnp.pad(W1a[:3, :], ((0, 5), (0, 0)))
    w2af = W2a[3:, :]
    w2axp = jnp.pad(W2a[:3, :], ((0, 5), (0, 0)))
    wh3p = jnp.pad(Wh3, ((0, 0), (0, 16 - ncls)))
    bh3p = jnp.pad(bh3, (0, 16 - ncls)).reshape(1, 16)

    f0, t1 = pl.pallas_call(
        _embed_t1_kernel,
        grid=(B, N // BLK),
        in_specs=[
            pl.BlockSpec((1, BLK, 8), lambda b, j: (b, j, 0)),
            _full((8, 64)), _full((1, 64)), _full((64, 128)),
            _full((8, 128)), _full((1, 128)),
        ],
        out_specs=[
            pl.BlockSpec((1, BLK, 64), lambda b, j: (b, j, 0)),
            pl.BlockSpec((1, BLK, 128), lambda b, j: (b, j, 0)),
        ],
        out_shape=[
            jax.ShapeDtypeStruct((B, N, 64), _F32),
            jax.ShapeDtypeStruct((B, N, 128), _F32),
        ],
    )(xp, wep, be.reshape(1, 64), w1af, w1axp, b1a.reshape(1, 128))

    CSZ = 128 if N % 128 == 0 else 32
    NC = N // CSZ
    nq1t = jnp.swapaxes(nq1, 1, 2)
    d2_full, cid_t = pl.pallas_call(
        functools.partial(_knn_dist_kernel, K, CSZ),
        grid=(B, S1 // QB),
        in_specs=[
            pl.BlockSpec((1, QB, 8), lambda b, q: (b, q, 0)),
            pl.BlockSpec((1, 8, N), lambda b, q: (b, 0, 0)),
            pl.BlockSpec((1, N, 8), lambda b, q: (b, 0, 0)),
            pl.BlockSpec((1, 8, QB), lambda b, q: (b, 0, q)),
        ],
        out_specs=[
            pl.BlockSpec((1, QB, N), lambda b, q: (b, q, 0)),
            pl.BlockSpec((1, K, QB), lambda b, q: (b, 0, q)),
        ],
        out_shape=[
            jax.ShapeDtypeStruct((B, S1, N), _F32),
            jax.ShapeDtypeStruct((B, K, S1), jnp.int32),
        ],
    )(nq1, xyzT, xp, nq1t)

    cid = jnp.swapaxes(cid_t, 1, 2)
    qrow = (jnp.arange(B, dtype=jnp.int32)[:, None, None] * S1
            + jnp.arange(S1, dtype=jnp.int32)[None, :, None])
    cand_rows = (qrow * NC + cid).reshape(B * S1 * K)
    cand = _gather_rows(d2_full.reshape(B * S1 * NC, CSZ), cand_rows)
    cand = cand.reshape(B, S1, K * CSZ)

    emat = jnp.repeat(jnp.eye(K, dtype=_F32), CSZ, axis=1)
    idx_knn1 = pl.pallas_call(
        functools.partial(_knn_select_kernel, K, CSZ, N),
        grid=(B,),
        in_specs=[
            pl.BlockSpec((1, S1, K * CSZ), lambda b: (b, 0, 0)),
            pl.BlockSpec((1, S1, K), lambda b: (b, 0, 0)),
            _full((K, K * CSZ)),
        ],
        out_specs=pl.BlockSpec((1, S1, K), lambda b: (b, 0, 0)),
        out_shape=jax.ShapeDtypeStruct((B, S1, K), jnp.int32),
    )(cand, cid, emat)

    flat_idx = (idx_knn1
                + (jnp.arange(B, dtype=jnp.int32) * N)[:, None, None]
                ).reshape(B * S1 * K)
    g1 = _gather_rows(t1.reshape(B * N, 128), flat_idx).reshape(B, S1 * K, 128)

    f1, t2 = pl.pallas_call(
        functools.partial(_sa1_finish_kernel, S1, K),
        grid=(B,),
        in_specs=[
            pl.BlockSpec((1, S1 * K, 128), lambda b: (b, 0, 0)),
            pl.BlockSpec((1, S1, 8), lambda b: (b, 0, 0)),
            _full((8, 128)), _full((128, 128)), _full((1, 128)),
            _full((128, 256)), _full((1, 256)), _full((8, 256)),
        ],
        out_specs=[
            pl.BlockSpec((1, S1, 128), lambda b: (b, 0, 0)),
            pl.BlockSpec((1, S1, 256), lambda b: (b, 0, 0)),
        ],
        out_shape=[
            jax.ShapeDtypeStruct((B, S1, 128), _F32),
            jax.ShapeDtypeStruct((B, S1, 256), _F32),
        ],
    )(g1, nq1, w1axp, W1b, b1b.reshape(1, 128), w2af, b2a.reshape(1, 256),
      w2axp)

    f1up = pl.pallas_call(
        functools.partial(_sa2_fp2_kernel, S1, S2, K),
        grid=(B,),
        in_specs=[
            pl.BlockSpec((1, S1, 8), lambda b: (b, 0, 0)),
            pl.BlockSpec((1, 8, S1), lambda b: (b, 0, 0)),
            pl.BlockSpec((1, S2, 8), lambda b: (b, 0, 0)),
            pl.BlockSpec((1, 8, S2), lambda b: (b, 0, 0)),
            pl.BlockSpec((1, S1, 256), lambda b: (b, 0, 0)),
            pl.BlockSpec((1, S1, 128), lambda b: (b, 0, 0)),
            _full((8, 256)), _full((256, 256)), _full((1, 256)),
            _full((128, 128)), _full((256, 128)), _full((1, 128)),
            _full((128, 128)), _full((1, 128)),
        ],
        out_specs=pl.BlockSpec((1, S1, 128), lambda b: (b, 0, 0)),
        out_shape=jax.ShapeDtypeStruct((B, S1, 128), _F32),
    )(nq1, x1t, nq2, x2t, t2, f1, w2axp, W2b, b2b.reshape(1, 256),
      Wf2a[:128, :], Wf2a[128:, :], bf2a.reshape(1, 128), Wf2b,
      bf2b.reshape(1, 128))

    f0up, gf = pl.pallas_call(
        _fp1_kernel,
        grid=(B, N // BLK),
        in_specs=[
            pl.BlockSpec((1, BLK, 8), lambda b, j: (b, j, 0)),
            pl.BlockSpec((1, BLK, 64), lambda b, j: (b, j, 0)),
            pl.BlockSpec((1, 8, S1), lambda b, j: (b, 0, 0)),
            pl.BlockSpec((1, S1, 128), lambda b, j: (b, 0, 0)),
            _full((64, 128)), _full((128, 128)), _full((1, 128)),
            _full((128, 128)), _full((1, 128)),
        ],
        out_specs=[
            pl.BlockSpec((1, BLK, 128), lambda b, j: (b, j, 0)),
            pl.BlockSpec((1, 8, 128), lambda b, j: (b, 0, 0)),
        ],
        out_shape=[
            jax.ShapeDtypeStruct((B, N, 128), _F32),
            jax.ShapeDtypeStruct((B, 8, 128), _F32),
        ],
    )(xp, f0, x1t, f1up, Wf1a[:64, :], Wf1a[64:, :], bf1a.reshape(1, 128),
      Wf1b, bf1b.reshape(1, 128))

    outp = pl.pallas_call(
        _head_kernel,
        grid=(B, N // BLK),
        in_specs=[
            pl.BlockSpec((1, BLK, 128), lambda b, j: (b, j, 0)),
            pl.BlockSpec((1, 8, 128), lambda b, j: (b, 0, 0)),
            _full((128, 128)), _full((128, 128)), _full((1, 128)),
            _full((128, 64)), _full((1, 64)), _full((64, 16)), _full((1, 16)),
        ],
        out_specs=pl.BlockSpec((1, BLK, 16), lambda b, j: (b, j, 0)),
        out_shape=jax.ShapeDtypeStruct((B, N, 16), _F32),
    )(f0up, gf, Wh1[:128, :], Wh1[128:, :], bh1.reshape(1, 128), Wh2,
      bh2.reshape(1, 64), wh3p, bh3p)

    return outp[..., :ncls]


# speedup vs baseline: 1.0827x; 1.0827x over previous
import functools

import jax
import jax.numpy as jnp
from jax import lax
from jax.experimental import pallas as pl
from jax.experimental.pallas import tpu as pltpu
from jax.experimental.pallas import tpu_sc as plsc

_F32 = jnp.float32
_BIG = 1e30


def _embed_t1_kernel(x_ref, wep_ref, be_ref, w1af_ref, w1axp_ref, b1a_ref,
                     f0_ref, t1_ref):
    xb = x_ref[0]
    f0 = jax.nn.relu(
        jnp.dot(xb, wep_ref[...], preferred_element_type=_F32) + be_ref[...])
    t1 = (jnp.dot(f0, w1af_ref[...], preferred_element_type=_F32)
          + jnp.dot(xb, w1axp_ref[...], preferred_element_type=_F32)
          + b1a_ref[...])
    f0_ref[0] = f0
    t1_ref[0] = t1


def _knn_dist_kernel(k, csz, nq_ref, pt_ref, xp_ref, nqt_ref, d2_ref, cid_ref):
    pt = pt_ref[0]
    pp = jnp.sum(pt * pt, axis=0, keepdims=True)
    d2 = jnp.dot(-2.0 * nq_ref[0], pt, preferred_element_type=_F32) + pp
    d2_ref[0] = d2

    xb = xp_ref[0]
    lane8 = lax.broadcasted_iota(jnp.int32, xb.shape, 1)
    ppt = jnp.sum(jnp.where(lane8 < 3, xb * xb, 0.0), axis=1, keepdims=True)
    d2t = jnp.dot(xb, -2.0 * nqt_ref[0], preferred_element_type=_F32) + ppt
    n, qb = d2t.shape
    c = n // csz
    cm = jnp.min(d2t.reshape(c, csz, qb), axis=1)

    sub_c = lax.broadcasted_iota(jnp.int32, (c, qb), 0)
    sub_k = lax.broadcasted_iota(jnp.int32, (k, qb), 0)

    def body(r, carry):
        cmc, acc = carry
        m = jnp.min(cmc, axis=0, keepdims=True)
        first = jnp.min(jnp.where(cmc <= m, sub_c, c), axis=0, keepdims=True)
        acc = jnp.where(sub_k == r, first, acc)
        cmc = jnp.where(sub_c == first, _BIG, cmc)
        return cmc, acc

    _, cid = lax.fori_loop(0, k, body, (cm, jnp.zeros((k, qb), jnp.int32)))

    def sbody(r, carry):
        ids, srt = carry
        m = jnp.min(ids, axis=0, keepdims=True)
        srt = jnp.where(sub_k == r, m, srt)
        ids = jnp.where(ids == m, 2 ** 30, ids)
        return ids, srt

    _, cid_sorted = lax.fori_loop(
        0, k, sbody, (cid, jnp.zeros((k, qb), jnp.int32)))
    cid_ref[0] = cid_sorted


def _knn_select_kernel(k, csz, n, cand_ref, cid_ref, e_ref, nq_ref, idx_ref):
    x1p = nq_ref[0]
    qq = jnp.sum(x1p * x1p, axis=1, keepdims=True)
    cand = jnp.maximum(cand_ref[0] + qq, 0.0)
    q, w = cand.shape
    base = jnp.dot(cid_ref[0].astype(_F32), e_ref[...],
                   preferred_element_type=_F32)
    iota_w = lax.broadcasted_iota(jnp.int32, (q, w), 1)
    orig = base.astype(jnp.int32) * csz + jnp.remainder(iota_w, csz)
    key = ((lax.bitcast_convert_type(cand, jnp.int32) & (~(w - 1))) | iota_w)
    lane_k = lax.broadcasted_iota(jnp.int32, (q, k), 1)

    def body(r, carry):
        keyc, acc = carry
        m = jnp.min(keyc, axis=1, keepdims=True)
        selm = keyc == m
        og = jnp.max(jnp.where(selm, orig, 0), axis=1, keepdims=True)
        acc = jnp.where(lane_k == r, og, acc)
        keyc = jnp.where(selm, 2147483647, keyc)
        return keyc, acc

    _, acc = lax.fori_loop(0, k, body, (key, jnp.zeros((q, k), jnp.int32)))
    idx_ref[0] = acc


def _gather_rows(table, idx):
    rows, depth = table.shape
    m = idx.shape[0]
    nw = 32
    ch = 128
    per_w = m // nw
    steps = per_w // ch
    mesh = plsc.VectorSubcoreMesh(core_axis_name="c", subcore_axis_name="s")

    @functools.partial(
        pl.kernel,
        mesh=mesh,
        out_type=jax.ShapeDtypeStruct((m, depth), _F32),
        scratch_types=[
            pltpu.VMEM((ch,), jnp.int32),
            pltpu.VMEM((ch, depth), _F32),
            pltpu.SemaphoreType.DMA,
        ],
    )
    def gk(tab_hbm, idx_hbm, out_hbm, idx_v, rows_v, sem):
        wid = lax.axis_index("s") * 2 + lax.axis_index("c")
        base = wid * per_w

        def body(j, carry):
            off = base + j * ch
            pltpu.sync_copy(idx_hbm.at[pl.ds(off, ch)], idx_v)
            pltpu.async_copy(tab_hbm.at[idx_v], rows_v, sem).wait()
            pltpu.sync_copy(rows_v, out_hbm.at[pl.ds(off, ch)])
            return carry

        lax.fori_loop(0, steps, body, 0)

    return gk(table, idx)


def _sa1_finish_kernel(s1, k, g_ref, nq_ref, w1axp_ref, w1b_ref, b1b_ref,
                       w2af_ref, b2a_ref, w2axp_ref, f1_ref, t2_ref):
    g = g_ref[0]
    pq = jnp.dot(nq_ref[0], w1axp_ref[...], preferred_element_type=_F32)
    h1 = jax.nn.relu(g.reshape(s1, k, 128) - pq[:, None, :]).reshape(s1 * k, 128)
    h2 = jax.nn.relu(
        jnp.dot(h1, w1b_ref[...], preferred_element_type=_F32) + b1b_ref[...])
    f1 = jnp.max(h2.reshape(s1, k, 128), axis=1)
    t2 = (jnp.dot(f1, w2af_ref[...], preferred_element_type=_F32)
          + b2a_ref[...]
          + jnp.dot(nq_ref[0], w2axp_ref[...], preferred_element_type=_F32))
    f1_ref[0] = f1
    t2_ref[0] = t2


def _top3_weights(d2f):
    q, s = d2f.shape
    iota_n = lax.broadcasted_iota(jnp.int32, (q, s), 1)
    inv = 1.0 / jnp.maximum(jnp.sqrt(d2f), 1e-10)
    w = jnp.zeros((q, s), _F32)
    d2c = d2f
    for _ in range(3):
        m = jnp.min(d2c, axis=1, keepdims=True)
        first = jnp.min(jnp.where(d2c <= m, iota_n, s), axis=1, keepdims=True)
        sel = iota_n == first
        w = jnp.where(sel, inv, w)
        d2c = jnp.where(sel, _BIG, d2c)
    return w / jnp.sum(w, axis=1, keepdims=True)


def _sa2_fp2_kernel(s1, s2, k, x1p_ref, x1t_ref, x2p_ref, x2t_ref, t2_ref,
                    f1_ref, w2axp_ref, w2b_ref, b2b_ref, wf2at_ref, wf2ab_ref,
                    bf2a_ref, wf2b_ref, bf2b_ref, f1up_ref):
    x1t = x1t_ref[0]
    pp1 = jnp.sum(x1t * x1t, axis=0, keepdims=True)
    d2 = jnp.dot(-2.0 * x2p_ref[0], x1t, preferred_element_type=_F32) + pp1
    iota_n = lax.broadcasted_iota(jnp.int32, (s2, s1), 1)
    pq2 = jnp.dot(x2p_ref[0], w2axp_ref[...], preferred_element_type=_F32)
    t2 = t2_ref[0]
    w2b = w2b_ref[...]
    b2b = b2b_ref[...]

    def body(r, carry):
        d2c, f2acc = carry
        m = jnp.min(d2c, axis=1, keepdims=True)
        first = jnp.min(jnp.where(d2c <= m, iota_n, s1), axis=1, keepdims=True)
        sel = (iota_n == first).astype(_F32)
        d2c = jnp.where(iota_n == first, _BIG, d2c)
        gr = jnp.dot(sel, t2, preferred_element_type=_F32)
        h1 = jax.nn.relu(gr - pq2)
        h2 = jax.nn.relu(jnp.dot(h1, w2b, preferred_element_type=_F32) + b2b)
        return d2c, jnp.maximum(f2acc, h2)

    _, f2 = lax.fori_loop(0, k, body, (d2, jnp.zeros((s2, 256), _F32)))

    x2t = x2t_ref[0]
    cc2 = jnp.sum(x2t * x2t, axis=0, keepdims=True)
    x1p = x1p_ref[0]
    qq1 = jnp.sum(x1p * x1p, axis=1, keepdims=True)
    d2f = jnp.maximum(
        jnp.dot(x1p, -2.0 * x2t, preferred_element_type=_F32) + cc2 + qq1, 0.0)
    w = _top3_weights(d2f)
    interp = jnp.dot(w, f2, preferred_element_type=_F32)
    xcat = jax.nn.relu(
        jnp.dot(f1_ref[0], wf2at_ref[...], preferred_element_type=_F32)
        + jnp.dot(interp, wf2ab_ref[...], preferred_element_type=_F32)
        + bf2a_ref[...])
    f1up = jax.nn.relu(
        jnp.dot(xcat, wf2b_ref[...], preferred_element_type=_F32) + bf2b_ref[...])
    f1up_ref[0] = f1up


def _fp1_kernel(xp_ref, f0_ref, x1t_ref, f1up_ref, wf1at_ref, wf1ab_ref,
                bf1a_ref, wf1b_ref, bf1b_ref, f0up_ref, gf_ref):
    xb = xp_ref[0]
    lane8 = lax.broadcasted_iota(jnp.int32, xb.shape, 1)
    qq = jnp.sum(jnp.where(lane8 < 3, xb * xb, 0.0), axis=1, keepdims=True)
    x1t = x1t_ref[0]
    cc1 = jnp.sum(x1t * x1t, axis=0, keepdims=True)
    d2 = jnp.maximum(
        jnp.dot(xb, -2.0 * x1t, preferred_element_type=_F32) + cc1 + qq, 0.0)
    w = _top3_weights(d2)
    interp = jnp.dot(w, f1up_ref[0], preferred_element_type=_F32)
    x1 = jax.nn.relu(
        jnp.dot(f0_ref[0], wf1at_ref[...], preferred_element_type=_F32)
        + jnp.dot(interp, wf1ab_ref[...], preferred_element_type=_F32)
        + bf1a_ref[...])
    f0up = jax.nn.relu(
        jnp.dot(x1, wf1b_ref[...], preferred_element_type=_F32) + bf1b_ref[...])
    f0up_ref[0] = f0up
    bm = jnp.broadcast_to(jnp.max(f0up, axis=0, keepdims=True), (8, 128))[None]

    @pl.when(pl.program_id(1) == 0)
    def _():
        gf_ref[...] = bm

    @pl.when(pl.program_id(1) != 0)
    def _():
        gf_ref[...] = jnp.maximum(gf_ref[...], bm)


def _head_kernel(f0up_ref, gf_ref, wh1t_ref, wh1b_ref, bh1_ref, wh2_ref,
                 bh2_ref, wh3_ref, bh3_ref, out_ref):
    bias = (jnp.dot(gf_ref[0, 0:1, :], wh1b_ref[...],
                    preferred_element_type=_F32)
            + bh1_ref[...])
    h1 = jax.nn.relu(
        jnp.dot(f0up_ref[0], wh1t_ref[...], preferred_element_type=_F32) + bias)
    h2 = jax.nn.relu(
        jnp.dot(h1, wh2_ref[...], preferred_element_type=_F32) + bh2_ref[...])
    out_ref[0] = (jnp.dot(h2, wh3_ref[...], preferred_element_type=_F32)
                  + bh3_ref[...])


def _full(shape):
    return pl.BlockSpec(shape, lambda *_: tuple(0 for _ in shape))


def kernel(x, seed, We, be, W1a, b1a, W1b, b1b, W2a, b2a, W2b, b2b,
           Wf2a, bf2a, Wf2b, bf2b, Wf1a, bf1a, Wf1b, bf1b,
           Wh1, bh1, Wh2, bh2, Wh3, bh3):
    B, N, C = x.shape
    S1, S2, K = 256, 64, 32
    BLK = 2048 if N % 2048 == 0 else N
    QB = 128
    ncls = Wh3.shape[1]

    xp = jnp.pad(x, ((0, 0), (0, 0), (0, 8 - C)))
    xyz = x[..., :3]

    idx_s1 = jnp.stack([
        jax.random.permutation(jax.random.key(seed + b * 17), N)[:S1]
        for b in range(B)], 0)
    new_xyz = jnp.take_along_axis(xyz, idx_s1[..., None], axis=1)
    nq1 = jnp.pad(new_xyz, ((0, 0), (0, 0), (0, 5)))
    xyzT = jnp.pad(jnp.swapaxes(xyz, 1, 2), ((0, 0), (0, 5), (0, 0)))

    idx_s2 = jnp.stack([
        jax.random.permutation(jax.random.key(seed + 1000 + b * 17), S1)[:S2]
        for b in range(B)], 0)
    new_xyz2 = jnp.take_along_axis(new_xyz, idx_s2[..., None], axis=1)
    nq2 = jnp.pad(new_xyz2, ((0, 0), (0, 0), (0, 5)))
    x1t = jnp.swapaxes(nq1, 1, 2)
    x2t = jnp.swapaxes(nq2, 1, 2)

    wep = jnp.pad(We, ((0, 8 - C), (0, 0)))
    w1af = W1a[3:, :]
    w1axp = jnp.pad(W1a[:3, :], ((0, 5), (0, 0)))
    w2af = W2a[3:, :]
    w2axp = jnp.pad(W2a[:3, :], ((0, 5), (0, 0)))
    wh3p = jnp.pad(Wh3, ((0, 0), (0, 16 - ncls)))
    bh3p = jnp.pad(bh3, (0, 16 - ncls)).reshape(1, 16)

    f0, t1 = pl.pallas_call(
        _embed_t1_kernel,
        grid=(B, N // BLK),
        in_specs=[
            pl.BlockSpec((1, BLK, 8), lambda b, j: (b, j, 0)),
            _full((8, 64)), _full((1, 64)), _full((64, 128)),
            _full((8, 128)), _full((1, 128)),
        ],
        out_specs=[
            pl.BlockSpec((1, BLK, 64), lambda b, j: (b, j, 0)),
            pl.BlockSpec((1, BLK, 128), lambda b, j: (b, j, 0)),
        ],
        out_shape=[
            jax.ShapeDtypeStruct((B, N, 64), _F32),
            jax.ShapeDtypeStruct((B, N, 128), _F32),
        ],
    )(xp, wep, be.reshape(1, 64), w1af, w1axp, b1a.reshape(1, 128))

    CSZ = 128 if N % 128 == 0 else 32
    NC = N // CSZ
    nq1t = jnp.swapaxes(nq1, 1, 2)
    d2_full, cid_t = pl.pallas_call(
        functools.partial(_knn_dist_kernel, K, CSZ),
        grid=(B, S1 // QB),
        in_specs=[
            pl.BlockSpec((1, QB, 8), lambda b, q: (b, q, 0)),
            pl.BlockSpec((1, 8, N), lambda b, q: (b, 0, 0)),
            pl.BlockSpec((1, N, 8), lambda b, q: (b, 0, 0)),
            pl.BlockSpec((1, 8, QB), lambda b, q: (b, 0, q)),
        ],
        out_specs=[
            pl.BlockSpec((1, QB, N), lambda b, q: (b, q, 0)),
            pl.BlockSpec((1, K, QB), lambda b, q: (b, 0, q)),
        ],
        out_shape=[
            jax.ShapeDtypeStruct((B, S1, N), _F32),
            jax.ShapeDtypeStruct((B, K, S1), jnp.int32),
        ],
    )(nq1, xyzT, xp, nq1t)

    cid = jnp.swapaxes(cid_t, 1, 2)
    qrow = (jnp.arange(B, dtype=jnp.int32)[:, None, None] * S1
            + jnp.arange(S1, dtype=jnp.int32)[None, :, None])
    cand_rows = (qrow * NC + cid).reshape(B * S1 * K)
    cand = _gather_rows(d2_full.reshape(B * S1 * NC, CSZ), cand_rows)
    cand = cand.reshape(B, S1, K * CSZ)

    emat = jnp.repeat(jnp.eye(K, dtype=_F32), CSZ, axis=1)
    idx_knn1 = pl.pallas_call(
        functools.partial(_knn_select_kernel, K, CSZ, N),
        grid=(B,),
        in_specs=[
            pl.BlockSpec((1, S1, K * CSZ), lambda b: (b, 0, 0)),
            pl.BlockSpec((1, S1, K), lambda b: (b, 0, 0)),
            _full((K, K * CSZ)),
            pl.BlockSpec((1, S1, 8), lambda b: (b, 0, 0)),
        ],
        out_specs=pl.BlockSpec((1, S1, K), lambda b: (b, 0, 0)),
        out_shape=jax.ShapeDtypeStruct((B, S1, K), jnp.int32),
    )(cand, cid, emat, nq1)

    flat_idx = (idx_knn1
                + (jnp.arange(B, dtype=jnp.int32) * N)[:, None, None]
                ).reshape(B * S1 * K)
    g1 = _gather_rows(t1.reshape(B * N, 128), flat_idx).reshape(B, S1 * K, 128)

    f1, t2 = pl.pallas_call(
        functools.partial(_sa1_finish_kernel, S1, K),
        grid=(B,),
        in_specs=[
            pl.BlockSpec((1, S1 * K, 128), lambda b: (b, 0, 0)),
            pl.BlockSpec((1, S1, 8), lambda b: (b, 0, 0)),
            _full((8, 128)), _full((128, 128)), _full((1, 128)),
            _full((128, 256)), _full((1, 256)), _full((8, 256)),
        ],
        out_specs=[
            pl.BlockSpec((1, S1, 128), lambda b: (b, 0, 0)),
            pl.BlockSpec((1, S1, 256), lambda b: (b, 0, 0)),
        ],
        out_shape=[
            jax.ShapeDtypeStruct((B, S1, 128), _F32),
            jax.ShapeDtypeStruct((B, S1, 256), _F32),
        ],
    )(g1, nq1, w1axp, W1b, b1b.reshape(1, 128), w2af, b2a.reshape(1, 256),
      w2axp)

    f1up = pl.pallas_call(
        functools.partial(_sa2_fp2_kernel, S1, S2, K),
        grid=(B,),
        in_specs=[
            pl.BlockSpec((1, S1, 8), lambda b: (b, 0, 0)),
            pl.BlockSpec((1, 8, S1), lambda b: (b, 0, 0)),
            pl.BlockSpec((1, S2, 8), lambda b: (b, 0, 0)),
            pl.BlockSpec((1, 8, S2), lambda b: (b, 0, 0)),
            pl.BlockSpec((1, S1, 256), lambda b: (b, 0, 0)),
            pl.BlockSpec((1, S1, 128), lambda b: (b, 0, 0)),
            _full((8, 256)), _full((256, 256)), _full((1, 256)),
            _full((128, 128)), _full((256, 128)), _full((1, 128)),
            _full((128, 128)), _full((1, 128)),
        ],
        out_specs=pl.BlockSpec((1, S1, 128), lambda b: (b, 0, 0)),
        out_shape=jax.ShapeDtypeStruct((B, S1, 128), _F32),
    )(nq1, x1t, nq2, x2t, t2, f1, w2axp, W2b, b2b.reshape(1, 256),
      Wf2a[:128, :], Wf2a[128:, :], bf2a.reshape(1, 128), Wf2b,
      bf2b.reshape(1, 128))

    f0up, gf = pl.pallas_call(
        _fp1_kernel,
        grid=(B, N // BLK),
        in_specs=[
            pl.BlockSpec((1, BLK, 8), lambda b, j: (b, j, 0)),
            pl.BlockSpec((1, BLK, 64), lambda b, j: (b, j, 0)),
            pl.BlockSpec((1, 8, S1), lambda b, j: (b, 0, 0)),
            pl.BlockSpec((1, S1, 128), lambda b, j: (b, 0, 0)),
            _full((64, 128)), _full((128, 128)), _full((1, 128)),
            _full((128, 128)), _full((1, 128)),
        ],
        out_specs=[
            pl.BlockSpec((1, BLK, 128), lambda b, j: (b, j, 0)),
            pl.BlockSpec((1, 8, 128), lambda b, j: (b, 0, 0)),
        ],
        out_shape=[
            jax.ShapeDtypeStruct((B, N, 128), _F32),
            jax.ShapeDtypeStruct((B, 8, 128), _F32),
        ],
    )(xp, f0, x1t, f1up, Wf1a[:64, :], Wf1a[64:, :], bf1a.reshape(1, 128),
      Wf1b, bf1b.reshape(1, 128))

    outp = pl.pallas_call(
        _head_kernel,
        grid=(B, N // BLK),
        in_specs=[
            pl.BlockSpec((1, BLK, 128), lambda b, j: (b, j, 0)),
            pl.BlockSpec((1, 8, 128), lambda b, j: (b, 0, 0)),
            _full((128, 128)), _full((128, 128)), _full((1, 128)),
            _full((128, 64)), _full((1, 64)), _full((64, 16)), _full((1, 16)),
        ],
        out_specs=pl.BlockSpec((1, BLK, 16), lambda b, j: (b, j, 0)),
        out_shape=jax.ShapeDtypeStruct((B, N, 16), _F32),
    )(f0up, gf, Wh1[:128, :], Wh1[128:, :], bh1.reshape(1, 128), Wh2,
      bh2.reshape(1, 64), wh3p, bh3p)

    return outp[..., :ncls]


# speedup vs baseline: 1.1380x; 1.0511x over previous
import functools

import jax
import jax.numpy as jnp
from jax import lax
from jax.experimental import pallas as pl
from jax.experimental.pallas import tpu as pltpu
from jax.experimental.pallas import tpu_sc as plsc

_F32 = jnp.float32
_BIG = 1e30


def _embed_t1_kernel(x_ref, wep_ref, be_ref, w1af_ref, w1axp_ref, b1a_ref,
                     f0_ref, t1_ref):
    xb = x_ref[0]
    f0 = jax.nn.relu(
        jnp.dot(xb, wep_ref[...], preferred_element_type=_F32) + be_ref[...])
    t1 = (jnp.dot(f0, w1af_ref[...], preferred_element_type=_F32)
          + jnp.dot(xb, w1axp_ref[...], preferred_element_type=_F32)
          + b1a_ref[...])
    f0_ref[0] = f0
    t1_ref[0] = t1


def _knn_dist_kernel(k, csz, nq_ref, pt_ref, xp_ref, nqt_ref, d2_ref, cid_ref):
    pt = pt_ref[0]
    pp = jnp.sum(pt * pt, axis=0, keepdims=True)
    d2 = jnp.dot(-2.0 * nq_ref[0], pt, preferred_element_type=_F32) + pp
    d2_ref[0] = d2

    xb = xp_ref[0]
    lane8 = lax.broadcasted_iota(jnp.int32, xb.shape, 1)
    ppt = jnp.sum(jnp.where(lane8 < 3, xb * xb, 0.0), axis=1, keepdims=True)
    d2t = jnp.dot(xb, -2.0 * nqt_ref[0], preferred_element_type=_F32) + ppt
    n, qb = d2t.shape
    c = n // csz
    cm = jnp.min(d2t.reshape(c, csz, qb), axis=1)

    sub_c = lax.broadcasted_iota(jnp.int32, (c, qb), 0)
    sub_k = lax.broadcasted_iota(jnp.int32, (k, qb), 0)

    def body(r, carry):
        cmc, acc = carry
        m = jnp.min(cmc, axis=0, keepdims=True)
        first = jnp.min(jnp.where(cmc <= m, sub_c, c), axis=0, keepdims=True)
        acc = jnp.where(sub_k == r, first, acc)
        cmc = jnp.where(sub_c == first, _BIG, cmc)
        return cmc, acc

    _, cid = lax.fori_loop(0, k, body, (cm, jnp.zeros((k, qb), jnp.int32)))

    def sbody(r, carry):
        ids, srt = carry
        m = jnp.min(ids, axis=0, keepdims=True)
        srt = jnp.where(sub_k == r, m, srt)
        ids = jnp.where(ids == m, 2 ** 30, ids)
        return ids, srt

    _, cid_sorted = lax.fori_loop(
        0, k, sbody, (cid, jnp.zeros((k, qb), jnp.int32)))
    cid_ref[0] = cid_sorted


def _knn_select_kernel(k, csz, n, cand_ref, cid_ref, e_ref, nq_ref, idx_ref):
    x1p = nq_ref[0]
    qq = jnp.sum(x1p * x1p, axis=1, keepdims=True)
    cand = jnp.maximum(cand_ref[0] + qq, 0.0)
    q, w = cand.shape
    base = jnp.dot(cid_ref[0].astype(_F32), e_ref[...],
                   preferred_element_type=_F32)
    iota_w = lax.broadcasted_iota(jnp.int32, (q, w), 1)
    orig = base.astype(jnp.int32) * csz + jnp.remainder(iota_w, csz)
    key = ((lax.bitcast_convert_type(cand, jnp.int32) & (~(w - 1))) | iota_w)
    lane_k = lax.broadcasted_iota(jnp.int32, (q, k), 1)

    keyc = key
    cols = []
    for _ in range(k):
        m = jnp.min(keyc, axis=1, keepdims=True)
        selm = keyc == m
        cols.append(jnp.max(jnp.where(selm, orig, 0), axis=1, keepdims=True))
        keyc = jnp.where(selm, 2147483647, keyc)
    idx_ref[0] = jnp.concatenate(cols, axis=1)


def _gather_rows(table, idx):
    rows, depth = table.shape
    m = idx.shape[0]
    nw = 32
    ch = 128
    per_w = m // nw
    steps = per_w // ch
    mesh = plsc.VectorSubcoreMesh(core_axis_name="c", subcore_axis_name="s")

    @functools.partial(
        pl.kernel,
        mesh=mesh,
        out_type=jax.ShapeDtypeStruct((m, depth), _F32),
        scratch_types=[
            pltpu.VMEM((ch,), jnp.int32),
            pltpu.VMEM((ch, depth), _F32),
            pltpu.SemaphoreType.DMA,
        ],
    )
    def gk(tab_hbm, idx_hbm, out_hbm, idx_v, rows_v, sem):
        wid = lax.axis_index("s") * 2 + lax.axis_index("c")
        base = wid * per_w

        def body(j, carry):
            off = base + j * ch
            pltpu.sync_copy(idx_hbm.at[pl.ds(off, ch)], idx_v)
            pltpu.async_copy(tab_hbm.at[idx_v], rows_v, sem).wait()
            pltpu.sync_copy(rows_v, out_hbm.at[pl.ds(off, ch)])
            return carry

        lax.fori_loop(0, steps, body, 0)

    return gk(table, idx)


def _sa1_finish_kernel(s1, k, g_ref, nq_ref, w1axp_ref, w1b_ref, b1b_ref,
                       w2af_ref, b2a_ref, w2axp_ref, f1_ref, t2_ref):
    g = g_ref[0]
    pq = jnp.dot(nq_ref[0], w1axp_ref[...], preferred_element_type=_F32)
    h1 = jax.nn.relu(g.reshape(s1, k, 128) - pq[:, None, :]).reshape(s1 * k, 128)
    h2 = jax.nn.relu(
        jnp.dot(h1, w1b_ref[...], preferred_element_type=_F32) + b1b_ref[...])
    f1 = jnp.max(h2.reshape(s1, k, 128), axis=1)
    t2 = (jnp.dot(f1, w2af_ref[...], preferred_element_type=_F32)
          + b2a_ref[...]
          + jnp.dot(nq_ref[0], w2axp_ref[...], preferred_element_type=_F32))
    f1_ref[0] = f1
    t2_ref[0] = t2


def _top3_weights(d2f):
    q, s = d2f.shape
    iota_n = lax.broadcasted_iota(jnp.int32, (q, s), 1)
    inv = 1.0 / jnp.maximum(jnp.sqrt(d2f), 1e-10)
    w = jnp.zeros((q, s), _F32)
    d2c = d2f
    for _ in range(3):
        m = jnp.min(d2c, axis=1, keepdims=True)
        first = jnp.min(jnp.where(d2c <= m, iota_n, s), axis=1, keepdims=True)
        sel = iota_n == first
        w = jnp.where(sel, inv, w)
        d2c = jnp.where(sel, _BIG, d2c)
    return w / jnp.sum(w, axis=1, keepdims=True)


def _sa2_fp2_kernel(s1, s2, k, x1p_ref, x1t_ref, x2p_ref, x2t_ref, t2_ref,
                    f1_ref, w2axp_ref, w2b_ref, b2b_ref, wf2at_ref, wf2ab_ref,
                    bf2a_ref, wf2b_ref, bf2b_ref, f1up_ref):
    x1t = x1t_ref[0]
    pp1 = jnp.sum(x1t * x1t, axis=0, keepdims=True)
    d2 = jnp.dot(-2.0 * x2p_ref[0], x1t, preferred_element_type=_F32) + pp1
    iota_n = lax.broadcasted_iota(jnp.int32, (s2, s1), 1)
    pq2 = jnp.dot(x2p_ref[0], w2axp_ref[...], preferred_element_type=_F32)
    t2 = t2_ref[0]
    w2b = w2b_ref[...]
    b2b = b2b_ref[...]

    def body(r, carry):
        d2c, f2acc = carry
        m = jnp.min(d2c, axis=1, keepdims=True)
        first = jnp.min(jnp.where(d2c <= m, iota_n, s1), axis=1, keepdims=True)
        sel = (iota_n == first).astype(_F32)
        d2c = jnp.where(iota_n == first, _BIG, d2c)
        gr = jnp.dot(sel, t2, preferred_element_type=_F32)
        h1 = jax.nn.relu(gr - pq2)
        h2 = jax.nn.relu(jnp.dot(h1, w2b, preferred_element_type=_F32) + b2b)
        return d2c, jnp.maximum(f2acc, h2)

    _, f2 = lax.fori_loop(0, k, body, (d2, jnp.zeros((s2, 256), _F32)))

    x2t = x2t_ref[0]
    cc2 = jnp.sum(x2t * x2t, axis=0, keepdims=True)
    x1p = x1p_ref[0]
    qq1 = jnp.sum(x1p * x1p, axis=1, keepdims=True)
    d2f = jnp.maximum(
        jnp.dot(x1p, -2.0 * x2t, preferred_element_type=_F32) + cc2 + qq1, 0.0)
    w = _top3_weights(d2f)
    interp = jnp.dot(w, f2, preferred_element_type=_F32)
    xcat = jax.nn.relu(
        jnp.dot(f1_ref[0], wf2at_ref[...], preferred_element_type=_F32)
        + jnp.dot(interp, wf2ab_ref[...], preferred_element_type=_F32)
        + bf2a_ref[...])
    f1up = jax.nn.relu(
        jnp.dot(xcat, wf2b_ref[...], preferred_element_type=_F32) + bf2b_ref[...])
    f1up_ref[0] = f1up


def _fp1_kernel(xp_ref, f0_ref, x1t_ref, f1up_ref, wf1at_ref, wf1ab_ref,
                bf1a_ref, wf1b_ref, bf1b_ref, f0up_ref, gf_ref):
    xb = xp_ref[0]
    lane8 = lax.broadcasted_iota(jnp.int32, xb.shape, 1)
    qq = jnp.sum(jnp.where(lane8 < 3, xb * xb, 0.0), axis=1, keepdims=True)
    x1t = x1t_ref[0]
    cc1 = jnp.sum(x1t * x1t, axis=0, keepdims=True)
    d2 = jnp.maximum(
        jnp.dot(xb, -2.0 * x1t, preferred_element_type=_F32) + cc1 + qq, 0.0)
    w = _top3_weights(d2)
    interp = jnp.dot(w, f1up_ref[0], preferred_element_type=_F32)
    x1 = jax.nn.relu(
        jnp.dot(f0_ref[0], wf1at_ref[...], preferred_element_type=_F32)
        + jnp.dot(interp, wf1ab_ref[...], preferred_element_type=_F32)
        + bf1a_ref[...])
    f0up = jax.nn.relu(
        jnp.dot(x1, wf1b_ref[...], preferred_element_type=_F32) + bf1b_ref[...])
    f0up_ref[0] = f0up
    bm = jnp.broadcast_to(jnp.max(f0up, axis=0, keepdims=True), (8, 128))[None]

    @pl.when(pl.program_id(1) == 0)
    def _():
        gf_ref[...] = bm

    @pl.when(pl.program_id(1) != 0)
    def _():
        gf_ref[...] = jnp.maximum(gf_ref[...], bm)


def _head_kernel(f0up_ref, gf_ref, wh1t_ref, wh1b_ref, bh1_ref, wh2_ref,
                 bh2_ref, wh3_ref, bh3_ref, out_ref):
    bias = (jnp.dot(gf_ref[0, 0:1, :], wh1b_ref[...],
                    preferred_element_type=_F32)
            + bh1_ref[...])
    h1 = jax.nn.relu(
        jnp.dot(f0up_ref[0], wh1t_ref[...], preferred_element_type=_F32) + bias)
    h2 = jax.nn.relu(
        jnp.dot(h1, wh2_ref[...], preferred_element_type=_F32) + bh2_ref[...])
    out_ref[0] = (jnp.dot(h2, wh3_ref[...], preferred_element_type=_F32)
                  + bh3_ref[...])


def _full(shape):
    return pl.BlockSpec(shape, lambda *_: tuple(0 for _ in shape))


def kernel(x, seed, We, be, W1a, b1a, W1b, b1b, W2a, b2a, W2b, b2b,
           Wf2a, bf2a, Wf2b, bf2b, Wf1a, bf1a, Wf1b, bf1b,
           Wh1, bh1, Wh2, bh2, Wh3, bh3):
    B, N, C = x.shape
    S1, S2, K = 256, 64, 32
    BLK = 2048 if N % 2048 == 0 else N
    QB = 128
    ncls = Wh3.shape[1]

    xp = jnp.pad(x, ((0, 0), (0, 0), (0, 8 - C)))
    xyz = x[..., :3]

    idx_s1 = jnp.stack([
        jax.random.permutation(jax.random.key(seed + b * 17), N)[:S1]
        for b in range(B)], 0)
    new_xyz = jnp.take_along_axis(xyz, idx_s1[..., None], axis=1)
    nq1 = jnp.pad(new_xyz, ((0, 0), (0, 0), (0, 5)))
    xyzT = jnp.pad(jnp.swapaxes(xyz, 1, 2), ((0, 0), (0, 5), (0, 0)))

    idx_s2 = jnp.stack([
        jax.random.permutation(jax.random.key(seed + 1000 + b * 17), S1)[:S2]
        for b in range(B)], 0)
    new_xyz2 = jnp.take_along_axis(new_xyz, idx_s2[..., None], axis=1)
    nq2 = jnp.pad(new_xyz2, ((0, 0), (0, 0), (0, 5)))
    x1t = jnp.swapaxes(nq1, 1, 2)
    x2t = jnp.swapaxes(nq2, 1, 2)

    wep = jnp.pad(We, ((0, 8 - C), (0, 0)))
    w1af = W1a[3:, :]
    w1axp = jnp.pad(W1a[:3, :], ((0, 5), (0, 0)))
    w2af = W2a[3:, :]
    w2axp = jnp.pad(W2a[:3, :], ((0, 5), (0, 0)))
    wh3p = jnp.pad(Wh3, ((0, 0), (0, 16 - ncls)))
    bh3p = jnp.pad(bh3, (0, 16 - ncls)).reshape(1, 16)

    f0, t1 = pl.pallas_call(
        _embed_t1_kernel,
        grid=(B, N // BLK),
        in_specs=[
            pl.BlockSpec((1, BLK, 8), lambda b, j: (b, j, 0)),
            _full((8, 64)), _full((1, 64)), _full((64, 128)),
            _full((8, 128)), _full((1, 128)),
        ],
        out_specs=[
            pl.BlockSpec((1, BLK, 64), lambda b, j: (b, j, 0)),
            pl.BlockSpec((1, BLK, 128), lambda b, j: (b, j, 0)),
        ],
        out_shape=[
            jax.ShapeDtypeStruct((B, N, 64), _F32),
            jax.ShapeDtypeStruct((B, N, 128), _F32),
        ],
    )(xp, wep, be.reshape(1, 64), w1af, w1axp, b1a.reshape(1, 128))

    CSZ = 128 if N % 128 == 0 else 32
    NC = N // CSZ
    nq1t = jnp.swapaxes(nq1, 1, 2)
    d2_full, cid_t = pl.pallas_call(
        functools.partial(_knn_dist_kernel, K, CSZ),
        grid=(B, S1 // QB),
        in_specs=[
            pl.BlockSpec((1, QB, 8), lambda b, q: (b, q, 0)),
            pl.BlockSpec((1, 8, N), lambda b, q: (b, 0, 0)),
            pl.BlockSpec((1, N, 8), lambda b, q: (b, 0, 0)),
            pl.BlockSpec((1, 8, QB), lambda b, q: (b, 0, q)),
        ],
        out_specs=[
            pl.BlockSpec((1, QB, N), lambda b, q: (b, q, 0)),
            pl.BlockSpec((1, K, QB), lambda b, q: (b, 0, q)),
        ],
        out_shape=[
            jax.ShapeDtypeStruct((B, S1, N), _F32),
            jax.ShapeDtypeStruct((B, K, S1), jnp.int32),
        ],
    )(nq1, xyzT, xp, nq1t)

    cid = jnp.swapaxes(cid_t, 1, 2)
    qrow = (jnp.arange(B, dtype=jnp.int32)[:, None, None] * S1
            + jnp.arange(S1, dtype=jnp.int32)[None, :, None])
    cand_rows = (qrow * NC + cid).reshape(B * S1 * K)
    cand = _gather_rows(d2_full.reshape(B * S1 * NC, CSZ), cand_rows)
    cand = cand.reshape(B, S1, K * CSZ)

    emat = jnp.repeat(jnp.eye(K, dtype=_F32), CSZ, axis=1)
    idx_knn1 = pl.pallas_call(
        functools.partial(_knn_select_kernel, K, CSZ, N),
        grid=(B,),
        in_specs=[
            pl.BlockSpec((1, S1, K * CSZ), lambda b: (b, 0, 0)),
            pl.BlockSpec((1, S1, K), lambda b: (b, 0, 0)),
            _full((K, K * CSZ)),
            pl.BlockSpec((1, S1, 8), lambda b: (b, 0, 0)),
        ],
        out_specs=pl.BlockSpec((1, S1, K), lambda b: (b, 0, 0)),
        out_shape=jax.ShapeDtypeStruct((B, S1, K), jnp.int32),
    )(cand, cid, emat, nq1)

    flat_idx = (idx_knn1
                + (jnp.arange(B, dtype=jnp.int32) * N)[:, None, None]
                ).reshape(B * S1 * K)
    g1 = _gather_rows(t1.reshape(B * N, 128), flat_idx).reshape(B, S1 * K, 128)

    f1, t2 = pl.pallas_call(
        functools.partial(_sa1_finish_kernel, S1, K),
        grid=(B,),
        in_specs=[
            pl.BlockSpec((1, S1 * K, 128), lambda b: (b, 0, 0)),
            pl.BlockSpec((1, S1, 8), lambda b: (b, 0, 0)),
            _full((8, 128)), _full((128, 128)), _full((1, 128)),
            _full((128, 256)), _full((1, 256)), _full((8, 256)),
        ],
        out_specs=[
            pl.BlockSpec((1, S1, 128), lambda b: (b, 0, 0)),
            pl.BlockSpec((1, S1, 256), lambda b: (b, 0, 0)),
        ],
        out_shape=[
            jax.ShapeDtypeStruct((B, S1, 128), _F32),
            jax.ShapeDtypeStruct((B, S1, 256), _F32),
        ],
    )(g1, nq1, w1axp, W1b, b1b.reshape(1, 128), w2af, b2a.reshape(1, 256),
      w2axp)

    f1up = pl.pallas_call(
        functools.partial(_sa2_fp2_kernel, S1, S2, K),
        grid=(B,),
        in_specs=[
            pl.BlockSpec((1, S1, 8), lambda b: (b, 0, 0)),
            pl.BlockSpec((1, 8, S1), lambda b: (b, 0, 0)),
            pl.BlockSpec((1, S2, 8), lambda b: (b, 0, 0)),
            pl.BlockSpec((1, 8, S2), lambda b: (b, 0, 0)),
            pl.BlockSpec((1, S1, 256), lambda b: (b, 0, 0)),
            pl.BlockSpec((1, S1, 128), lambda b: (b, 0, 0)),
            _full((8, 256)), _full((256, 256)), _full((1, 256)),
            _full((128, 128)), _full((256, 128)), _full((1, 128)),
            _full((128, 128)), _full((1, 128)),
        ],
        out_specs=pl.BlockSpec((1, S1, 128), lambda b: (b, 0, 0)),
        out_shape=jax.ShapeDtypeStruct((B, S1, 128), _F32),
    )(nq1, x1t, nq2, x2t, t2, f1, w2axp, W2b, b2b.reshape(1, 256),
      Wf2a[:128, :], Wf2a[128:, :], bf2a.reshape(1, 128), Wf2b,
      bf2b.reshape(1, 128))

    f0up, gf = pl.pallas_call(
        _fp1_kernel,
        grid=(B, N // BLK),
        in_specs=[
            pl.BlockSpec((1, BLK, 8), lambda b, j: (b, j, 0)),
            pl.BlockSpec((1, BLK, 64), lambda b, j: (b, j, 0)),
            pl.BlockSpec((1, 8, S1), lambda b, j: (b, 0, 0)),
            pl.BlockSpec((1, S1, 128), lambda b, j: (b, 0, 0)),
            _full((64, 128)), _full((128, 128)), _full((1, 128)),
            _full((128, 128)), _full((1, 128)),
        ],
        out_specs=[
            pl.BlockSpec((1, BLK, 128), lambda b, j: (b, j, 0)),
            pl.BlockSpec((1, 8, 128), lambda b, j: (b, 0, 0)),
        ],
        out_shape=[
            jax.ShapeDtypeStruct((B, N, 128), _F32),
            jax.ShapeDtypeStruct((B, 8, 128), _F32),
        ],
    )(xp, f0, x1t, f1up, Wf1a[:64, :], Wf1a[64:, :], bf1a.reshape(1, 128),
      Wf1b, bf1b.reshape(1, 128))

    outp = pl.pallas_call(
        _head_kernel,
        grid=(B, N // BLK),
        in_specs=[
            pl.BlockSpec((1, BLK, 128), lambda b, j: (b, j, 0)),
            pl.BlockSpec((1, 8, 128), lambda b, j: (b, 0, 0)),
            _full((128, 128)), _full((128, 128)), _full((1, 128)),
            _full((128, 64)), _full((1, 64)), _full((64, 16)), _full((1, 16)),
        ],
        out_specs=pl.BlockSpec((1, BLK, 16), lambda b, j: (b, j, 0)),
        out_shape=jax.ShapeDtypeStruct((B, N, 16), _F32),
    )(f0up, gf, Wh1[:128, :], Wh1[128:, :], bh1.reshape(1, 128), Wh2,
      bh2.reshape(1, 64), wh3p, bh3p)

    return outp[..., :ncls]
